# baseline (speedup 1.0000x reference)
import jax
import jax.numpy as jnp
from jax.experimental import pallas as pl

K = 5
B = 16
elu = jax.nn.elu


def _bmm(P, H):
    Bb, M, N = P.shape
    F = H.shape[2]

    def body(p_ref, h_ref, o_ref):
        o_ref[...] = jnp.dot(p_ref[0], h_ref[0],
                             preferred_element_type=jnp.float32)[None]

    return pl.pallas_call(
        body,
        grid=(Bb,),
        in_specs=[pl.BlockSpec((1, M, N), lambda b: (b, 0, 0)),
                  pl.BlockSpec((1, N, F), lambda b: (b, 0, 0))],
        out_specs=pl.BlockSpec((1, M, F), lambda b: (b, 0, 0)),
        out_shape=jax.ShapeDtypeStruct((Bb, M, F), jnp.float32),
    )(P, H)


def _spline_conv(x, edge_index, pseudo, W, root, bias):
    src = edge_index[0]
    dst = edge_index[1]
    x_j = jnp.take(x, src, axis=0)
    v = jnp.clip(pseudo, 0.0, 1.0) * (K - 1)
    v = jnp.minimum(v, K - 1 - 1e-6)
    bot = jnp.floor(v)
    boti = bot.astype(jnp.int32)
    frac = v - bot
    out = jnp.zeros((x_j.shape[0], W.shape[2]), dtype=x.dtype)
    for s0 in (0, 1):
        for s1 in (0, 1):
            for s2 in (0, 1):
                b = ((frac[:, 0] if s0 else 1.0 - frac[:, 0])
                     * (frac[:, 1] if s1 else 1.0 - frac[:, 1])
                     * (frac[:, 2] if s2 else 1.0 - frac[:, 2]))
                idx = (boti[:, 0] + s0) + (boti[:, 1] + s1) * K + (boti[:, 2] + s2) * (K * K)
                Wg = jnp.take(W, idx, axis=0)
                out = out + b[:, None] * jnp.einsum('ei,eio->eo', x_j, Wg)
    agg = jnp.zeros((x.shape[0], W.shape[2]), dtype=x.dtype).at[dst].add(out)
    return agg + x @ root + bias


def kernel(x, edge_attr, bg1_edge_attr, bg2_edge_attr, bg3_edge_attr, bg4_edge_attr, P01, P12, P23, P34, Pn1, P1n, P10, P21, P32, P43, eps, W1, R1, b1, W2, R2, b2, W3, R3, b3, W4, R4, b4, W5, R5, b5, fce1_w, fce1_b, fce21_w, fce21_b, fce22_w, fce22_b, fcd3_w, fcd3_b, fcd4_w, fcd4_b, dW5, dR5, db5, dW4, dR4, db4, dW3, dR3, db3, dW2, dR2, db2, dW1, dR1, db1, edge_index, bg1_edge_index, bg2_edge_index, bg3_edge_index, bg4_edge_index):
    h = elu(_spline_conv(x, edge_index, edge_attr, W1, R1, b1))
    h = _bmm(P01, h.reshape(B, -1, 16)).reshape(-1, 16)
    h = elu(_spline_conv(h, bg1_edge_index, bg1_edge_attr, W2, R2, b2))
    h = _bmm(P12, h.reshape(B, -1, 16)).reshape(-1, 16)
    h = elu(_spline_conv(h, bg2_edge_index, bg2_edge_attr, W3, R3, b3))
    h = _bmm(P23, h.reshape(B, -1, 16)).reshape(-1, 16)
    h = elu(_spline_conv(h, bg3_edge_index, bg3_edge_attr, W4, R4, b4))
    h = _bmm(P34, h.reshape(B, -1, 32)).reshape(-1, 32)
    h = elu(_spline_conv(h, bg4_edge_index, bg4_edge_attr, W5, R5, b5))
    h = _bmm(Pn1, h.reshape(B, -1, 32)).reshape(B, 32)
    h = elu(h @ fce1_w.T + fce1_b)
    mu = h @ fce21_w.T + fce21_b
    logvar = h @ fce22_w.T + fce22_b
    z = eps * jnp.exp(0.5 * logvar) + mu
    d = elu(z @ fcd3_w.T + fcd3_b)
    d = elu(d @ fcd4_w.T + fcd4_b)
    d = _bmm(P1n, d.reshape(B, 1, 32)).reshape(-1, 32)
    d = elu(_spline_conv(d, bg4_edge_index, bg4_edge_attr, dW5, dR5, db5))
    d = _bmm(P43, d.reshape(B, -1, 32)).reshape(-1, 32)
    d = elu(_spline_conv(d, bg3_edge_index, bg3_edge_attr, dW4, dR4, db4))
    d = _bmm(P32, d.reshape(B, -1, 16)).reshape(-1, 16)
    d = elu(_spline_conv(d, bg2_edge_index, bg2_edge_attr, dW3, dR3, db3))
    d = _bmm(P21, d.reshape(B, -1, 16)).reshape(-1, 16)
    d = elu(_spline_conv(d, bg1_edge_index, bg1_edge_attr, dW2, dR2, db2))
    d = _bmm(P10, d.reshape(B, -1, 16)).reshape(-1, 16)
    d = elu(_spline_conv(d, edge_index, edge_attr, dW1, dR1, db1))
    recon = d.reshape(B, -1)
    return (recon, mu, logvar)


# speedup vs baseline: 30.9481x; 30.9481x over previous
import functools

import numpy as np
import jax
import jax.numpy as jnp
from jax import lax
from jax.experimental import pallas as pl
from jax.experimental.pallas import tpu as pltpu, tpu_sc as plsc

K = 5
B = 16
NC, NS = 2, 16
NW = NC * NS


@functools.lru_cache(maxsize=None)
def _sc_gather(E, N, D):
    total_rows = E // 128
    rpw = max(1, total_rows // NW)
    active = total_rows // rpw
    CH = min(rpw, 8)
    n_super = rpw // CH
    mesh = plsc.VectorSubcoreMesh(core_axis_name="c", subcore_axis_name="s")

    @functools.partial(
        pl.kernel,
        out_type=jax.ShapeDtypeStruct((E, D), jnp.float32),
        mesh=mesh,
        scratch_types=[
            pltpu.VMEM((CH, 128), jnp.int32),
            pltpu.VMEM((CH * 128, D), jnp.float32),
            pltpu.SemaphoreType.DMA,
        ],
        compiler_params=pltpu.CompilerParams(use_tc_tiling_on_sc=False),
    )
    def kern(table, idx, out, idx_v, rows_v, sem):
        wid = lax.axis_index("s") * NC + lax.axis_index("c")

        @pl.when(wid < active)
        def _():
            for sc_i in range(n_super):
                r0 = wid * rpw + sc_i * CH
                pltpu.sync_copy(idx.at[pl.ds(r0, CH)], idx_v)
                cps = [
                    pltpu.async_copy(
                        table.at[idx_v.at[j]],
                        rows_v.at[pl.ds(j * 128, 128)], sem)
                    for j in range(CH)
                ]
                for cp in cps:
                    cp.wait()
                pltpu.sync_copy(rows_v, out.at[pl.ds(r0 * 128, CH * 128)])

    return kern


@functools.lru_cache(maxsize=None)
def _sc_scatter(E, N, D):
    total_rows = E // 128
    rpw = max(1, total_rows // NW)
    active = total_rows // rpw
    CH = min(rpw, 8)
    n_super = rpw // CH
    rows_t = N // NS
    mesh = plsc.VectorSubcoreMesh(core_axis_name="c", subcore_axis_name="s")

    @functools.partial(
        pl.kernel,
        out_type=jax.ShapeDtypeStruct((2, N, D), jnp.float32),
        mesh=mesh,
        scratch_types=[
            pltpu.VMEM((CH, 128), jnp.int32),
            pltpu.VMEM((CH * 128, D), jnp.float32),
            pltpu.SemaphoreType.DMA,
            pltpu.VMEM_SHARED((N, D), jnp.float32),
        ],
        compiler_params=pltpu.CompilerParams(use_tc_tiling_on_sc=False),
    )
    def kern(upd, idx, zeros, out, idx_v, upd_v, sem, shared):
        c = lax.axis_index("c")
        s = lax.axis_index("s")
        wid = s * NC + c
        pltpu.sync_copy(zeros.at[pl.ds(s * rows_t, rows_t)],
                        shared.at[pl.ds(s * rows_t, rows_t)])
        plsc.subcore_barrier()

        @pl.when(wid < active)
        def _():
            for sc_i in range(n_super):
                r0 = wid * rpw + sc_i * CH
                pltpu.sync_copy(idx.at[pl.ds(r0, CH)], idx_v)
                pltpu.sync_copy(upd.at[pl.ds(r0 * 128, CH * 128)], upd_v)
                cps = [
                    pltpu.async_copy(
                        upd_v.at[pl.ds(j * 128, 128)],
                        shared.at[idx_v.at[j]], sem, add=True)
                    for j in range(CH)
                ]
                for cp in cps:
                    cp.wait()

        plsc.subcore_barrier()
        pltpu.sync_copy(shared.at[pl.ds(s * rows_t, rows_t)],
                        out.at[c, pl.ds(s * rows_t, rows_t)])

    return kern


@functools.lru_cache(maxsize=None)
def _edge_consts(din, din_p, dout, dout_p):
    rep = []
    for d in range(3):
        m = np.zeros((8, 128), np.float32)
        for k in range(125):
            dig = (k, k // 5, k // 25)[d] % 5
            m[dig, k] = 1.0
        rep.append(m)
    repx = np.zeros((din_p, din * dout), np.float32)
    red = np.zeros((din * dout, dout_p), np.float32)
    for i in range(din):
        repx[i, i * dout:(i + 1) * dout] = 1.0
        for o in range(dout):
            red[i * dout + o, o] = 1.0
    return rep[0], rep[1], rep[2], repx, red


@functools.lru_cache(maxsize=None)
def _tc_edge(E, din, din_p, dout, dout_p):
    Te = min(E, 2048)
    grid = E // Te
    dio = din * dout

    def body(xj_ref, attr_ref, wf_ref, r0_ref, r1_ref, r2_ref, rx_ref,
             rd_ref, out_ref):
        a = attr_ref[...]
        v = jnp.minimum(jnp.clip(a, 0.0, 1.0) * (K - 1), K - 1 - 1e-6)
        bot = jnp.floor(v)
        boti = bot.astype(jnp.int32)
        frac = v - bot
        ii = lax.broadcasted_iota(jnp.int32, (Te, 8), 1)
        cs = []
        for d in range(3):
            bd = boti[:, d:d + 1]
            fd = frac[:, d:d + 1]
            cs.append(jnp.where(ii == bd, 1.0 - fd, 0.0)
                      + jnp.where(ii == bd + 1, fd, 0.0))
        f32 = jnp.float32
        C = (jnp.dot(cs[0], r0_ref[...], preferred_element_type=f32)
             * jnp.dot(cs[1], r1_ref[...], preferred_element_type=f32)
             * jnp.dot(cs[2], r2_ref[...], preferred_element_type=f32))
        weff = jnp.dot(C, wf_ref[...], preferred_element_type=f32)
        xrep = jnp.dot(xj_ref[...], rx_ref[...], preferred_element_type=f32)
        out_ref[...] = jnp.dot(xrep * weff, rd_ref[...],
                               preferred_element_type=f32)

    return pl.pallas_call(
        body,
        grid=(grid,),
        in_specs=[
            pl.BlockSpec((Te, din_p), lambda i: (i, 0)),
            pl.BlockSpec((Te, 3), lambda i: (i, 0)),
            pl.BlockSpec((128, dio), lambda i: (0, 0)),
            pl.BlockSpec((8, 128), lambda i: (0, 0)),
            pl.BlockSpec((8, 128), lambda i: (0, 0)),
            pl.BlockSpec((8, 128), lambda i: (0, 0)),
            pl.BlockSpec((din_p, dio), lambda i: (0, 0)),
            pl.BlockSpec((dio, dout_p), lambda i: (0, 0)),
        ],
        out_specs=pl.BlockSpec((Te, dout_p), lambda i: (i, 0)),
        out_shape=jax.ShapeDtypeStruct((E, dout_p), jnp.float32),
    )


@functools.lru_cache(maxsize=None)
def _tc_node(N, din, dout, dout_p):
    Tn = min(N, 4096)
    grid = N // Tn

    def body(p_ref, x_ref, root_ref, bias_ref, out_ref):
        agg = p_ref[0] + p_ref[1]
        o = (agg[:, :dout]
             + jnp.dot(x_ref[...], root_ref[...],
                       preferred_element_type=jnp.float32)
             + bias_ref[...])
        out_ref[...] = jnp.where(o > 0.0, o, jnp.exp(jnp.minimum(o, 0.0)) - 1.0)

    return pl.pallas_call(
        body,
        grid=(grid,),
        in_specs=[
            pl.BlockSpec((2, Tn, dout_p), lambda i: (0, i, 0)),
            pl.BlockSpec((Tn, din), lambda i: (i, 0)),
            pl.BlockSpec((din, dout), lambda i: (0, 0)),
            pl.BlockSpec((1, dout), lambda i: (0, 0)),
        ],
        out_specs=pl.BlockSpec((Tn, dout), lambda i: (i, 0)),
        out_shape=jax.ShapeDtypeStruct((N, dout), jnp.float32),
    )


def _spline_sc(h, src2d, dst2d, attr, W, root, bias):
    N, din = h.shape
    dout = W.shape[2]
    E = attr.shape[0]
    din_p = max(din, 16)
    dout_p = max(dout, 16)
    table = h if din == din_p else jnp.pad(h, ((0, 0), (0, din_p - din)))
    xj = _sc_gather(E, N, din_p)(table, src2d)
    wflat = jnp.pad(W.reshape(125, din * dout), ((0, 3), (0, 0)))
    consts = _edge_consts(din, din_p, dout, dout_p)
    oute = _tc_edge(E, din, din_p, dout, dout_p)(xj, attr, wflat, *consts)
    zeros = jnp.zeros((N, dout_p), jnp.float32)
    parts = _sc_scatter(E, N, dout_p)(oute, dst2d, zeros)
    return _tc_node(N, din, dout, dout_p)(parts, h, root, bias.reshape(1, dout))


def _bmm(P, H):
    Bb, M, N = P.shape
    F = H.shape[2]

    def body(p_ref, h_ref, o_ref):
        o_ref[...] = jnp.dot(p_ref[0], h_ref[0],
                             preferred_element_type=jnp.float32)[None]

    return pl.pallas_call(
        body,
        grid=(Bb,),
        in_specs=[pl.BlockSpec((1, M, N), lambda b: (b, 0, 0)),
                  pl.BlockSpec((1, N, F), lambda b: (b, 0, 0))],
        out_specs=pl.BlockSpec((1, M, F), lambda b: (b, 0, 0)),
        out_shape=jax.ShapeDtypeStruct((Bb, M, F), jnp.float32),
    )(P, H)


def kernel(x, edge_attr, bg1_edge_attr, bg2_edge_attr, bg3_edge_attr, bg4_edge_attr, P01, P12, P23, P34, Pn1, P1n, P10, P21, P32, P43, eps, W1, R1, b1, W2, R2, b2, W3, R3, b3, W4, R4, b4, W5, R5, b5, fce1_w, fce1_b, fce21_w, fce21_b, fce22_w, fce22_b, fcd3_w, fcd3_b, fcd4_w, fcd4_b, dW5, dR5, db5, dW4, dR4, db4, dW3, dR3, db3, dW2, dR2, db2, dW1, dR1, db1, edge_index, bg1_edge_index, bg2_edge_index, bg3_edge_index, bg4_edge_index):
    elu = jax.nn.elu
    eis = []
    for ei in (edge_index, bg1_edge_index, bg2_edge_index, bg3_edge_index,
               bg4_edge_index):
        eis.append((ei[0].reshape(-1, 128), ei[1].reshape(-1, 128)))

    h = _spline_sc(x, *eis[0], edge_attr, W1, R1, b1)
    h = _bmm(P01, h.reshape(B, -1, 16)).reshape(-1, 16)
    h = _spline_sc(h, *eis[1], bg1_edge_attr, W2, R2, b2)
    h = _bmm(P12, h.reshape(B, -1, 16)).reshape(-1, 16)
    h = _spline_sc(h, *eis[2], bg2_edge_attr, W3, R3, b3)
    h = _bmm(P23, h.reshape(B, -1, 16)).reshape(-1, 16)
    h = _spline_sc(h, *eis[3], bg3_edge_attr, W4, R4, b4)
    h = _bmm(P34, h.reshape(B, -1, 32)).reshape(-1, 32)
    h = _spline_sc(h, *eis[4], bg4_edge_attr, W5, R5, b5)
    h = _bmm(Pn1, h.reshape(B, -1, 32)).reshape(B, 32)
    h = elu(h @ fce1_w.T + fce1_b)
    mu = h @ fce21_w.T + fce21_b
    logvar = h @ fce22_w.T + fce22_b
    z = eps * jnp.exp(0.5 * logvar) + mu
    d = elu(z @ fcd3_w.T + fcd3_b)
    d = elu(d @ fcd4_w.T + fcd4_b)
    d = _bmm(P1n, d.reshape(B, 1, 32)).reshape(-1, 32)
    d = _spline_sc(d, *eis[4], bg4_edge_attr, dW5, dR5, db5)
    d = _bmm(P43, d.reshape(B, -1, 32)).reshape(-1, 32)
    d = _spline_sc(d, *eis[3], bg3_edge_attr, dW4, dR4, db4)
    d = _bmm(P32, d.reshape(B, -1, 16)).reshape(-1, 16)
    d = _spline_sc(d, *eis[2], bg2_edge_attr, dW3, dR3, db3)
    d = _bmm(P21, d.reshape(B, -1, 16)).reshape(-1, 16)
    d = _spline_sc(d, *eis[1], bg1_edge_attr, dW2, dR2, db2)
    d = _bmm(P10, d.reshape(B, -1, 16)).reshape(-1, 16)
    d = _spline_sc(d, *eis[0], edge_attr, dW1, dR1, db1)
    recon = d.reshape(B, -1)
    return (recon, mu, logvar)


# speedup vs baseline: 32.6704x; 1.0557x over previous
import functools

import numpy as np
import jax
import jax.numpy as jnp
from jax import lax
from jax.experimental import pallas as pl
from jax.experimental.pallas import tpu as pltpu, tpu_sc as plsc

K = 5
B = 16
NC, NS = 2, 16
NW = NC * NS


@functools.lru_cache(maxsize=None)
def _sc_gather(E, N, D):
    total_rows = E // 128
    rpw = max(1, total_rows // NW)
    active = total_rows // rpw
    CH = min(rpw, 16 if D <= 16 else 8)
    n_super = rpw // CH
    mesh = plsc.VectorSubcoreMesh(core_axis_name="c", subcore_axis_name="s")

    @functools.partial(
        pl.kernel,
        out_type=jax.ShapeDtypeStruct((E, D), jnp.float32),
        mesh=mesh,
        scratch_types=[
            pltpu.VMEM((2, CH, 128), jnp.int32),
            pltpu.VMEM((2, CH * 128, D), jnp.float32),
            pltpu.SemaphoreType.DMA,
            pltpu.SemaphoreType.DMA,
        ],
        compiler_params=pltpu.CompilerParams(use_tc_tiling_on_sc=False),
    )
    def kern(table, idx, out, idx_v, rows_v, gsem, wsem):
        wid = lax.axis_index("s") * NC + lax.axis_index("c")

        @pl.when(wid < active)
        def _():
            wbs = [None, None]
            for i in range(n_super):
                b = i & 1
                if wbs[b] is not None:
                    wbs[b].wait()
                r0 = wid * rpw + i * CH
                pltpu.sync_copy(idx.at[pl.ds(r0, CH)], idx_v.at[b])
                cps = [
                    pltpu.async_copy(
                        table.at[idx_v.at[b, j]],
                        rows_v.at[b, pl.ds(j * 128, 128)], gsem)
                    for j in range(CH)
                ]
                for cp in cps:
                    cp.wait()
                wbs[b] = pltpu.async_copy(
                    rows_v.at[b], out.at[pl.ds(r0 * 128, CH * 128)], wsem)
            for wb in wbs:
                if wb is not None:
                    wb.wait()

    return kern


@functools.lru_cache(maxsize=None)
def _sc_scatter(E, N, D):
    total_rows = E // 128
    rpw = max(1, total_rows // NW)
    active = total_rows // rpw
    CH = min(rpw, 16 if D <= 16 else 8)
    n_super = rpw // CH
    rows_t = N // NS
    mesh = plsc.VectorSubcoreMesh(core_axis_name="c", subcore_axis_name="s")

    @functools.partial(
        pl.kernel,
        out_type=jax.ShapeDtypeStruct((2, N, D), jnp.float32),
        mesh=mesh,
        scratch_types=[
            pltpu.VMEM((2, CH, 128), jnp.int32),
            pltpu.VMEM((2, CH * 128, D), jnp.float32),
            pltpu.SemaphoreType.DMA,
            pltpu.SemaphoreType.DMA,
            pltpu.VMEM_SHARED((N, D), jnp.float32),
        ],
        compiler_params=pltpu.CompilerParams(use_tc_tiling_on_sc=False),
    )
    def kern(upd, idx, zeros, out, idx_v, upd_v, ssem, psem, shared):
        c = lax.axis_index("c")
        s = lax.axis_index("s")
        wid = s * NC + c
        pltpu.sync_copy(zeros.at[pl.ds(s * rows_t, rows_t)],
                        shared.at[pl.ds(s * rows_t, rows_t)])
        plsc.subcore_barrier()

        @pl.when(wid < active)
        def _():
            def stage(i):
                b = i & 1
                r0 = wid * rpw + i * CH
                return (
                    pltpu.async_copy(idx.at[pl.ds(r0, CH)], idx_v.at[b], psem),
                    pltpu.async_copy(upd.at[pl.ds(r0 * 128, CH * 128)],
                                     upd_v.at[b], psem),
                )

            nxt = stage(0)
            for i in range(n_super):
                b = i & 1
                for cp in nxt:
                    cp.wait()
                if i + 1 < n_super:
                    nxt = stage(i + 1)
                cps = [
                    pltpu.async_copy(
                        upd_v.at[b, pl.ds(j * 128, 128)],
                        shared.at[idx_v.at[b, j]], ssem, add=True)
                    for j in range(CH)
                ]
                for cp in cps:
                    cp.wait()

        plsc.subcore_barrier()
        pltpu.sync_copy(shared.at[pl.ds(s * rows_t, rows_t)],
                        out.at[c, pl.ds(s * rows_t, rows_t)])

    return kern


@functools.lru_cache(maxsize=None)
def _edge_consts(din, din_p, dout, dout_p):
    rep = []
    for d in range(3):
        m = np.zeros((8, 128), np.float32)
        for k in range(125):
            dig = (k, k // 5, k // 25)[d] % 5
            m[dig, k] = 1.0
        rep.append(m)
    repx = np.zeros((din_p, din * dout), np.float32)
    red = np.zeros((din * dout, dout_p), np.float32)
    for i in range(din):
        repx[i, i * dout:(i + 1) * dout] = 1.0
        for o in range(dout):
            red[i * dout + o, o] = 1.0
    return rep[0], rep[1], rep[2], repx, red


@functools.lru_cache(maxsize=None)
def _tc_edge(E, din, din_p, dout, dout_p):
    Te = E // 32 if E >= 32768 else min(E, 2048)
    grid = E // Te
    dio = din * dout

    def body(xj_ref, attr_ref, wf_ref, r0_ref, r1_ref, r2_ref, rx_ref,
             rd_ref, out_ref):
        a = attr_ref[...]
        v = jnp.minimum(jnp.clip(a, 0.0, 1.0) * (K - 1), K - 1 - 1e-6)
        bot = jnp.floor(v)
        boti = bot.astype(jnp.int32)
        frac = v - bot
        ii = lax.broadcasted_iota(jnp.int32, (Te, 8), 1)
        cs = []
        for d in range(3):
            bd = boti[:, d:d + 1]
            fd = frac[:, d:d + 1]
            cs.append(jnp.where(ii == bd, 1.0 - fd, 0.0)
                      + jnp.where(ii == bd + 1, fd, 0.0))
        f32 = jnp.float32
        C = (jnp.dot(cs[0], r0_ref[...], preferred_element_type=f32)
             * jnp.dot(cs[1], r1_ref[...], preferred_element_type=f32)
             * jnp.dot(cs[2], r2_ref[...], preferred_element_type=f32))
        weff = jnp.dot(C, wf_ref[...], preferred_element_type=f32)
        xrep = jnp.dot(xj_ref[...], rx_ref[...], preferred_element_type=f32)
        out_ref[...] = jnp.dot(xrep * weff, rd_ref[...],
                               preferred_element_type=f32)

    return pl.pallas_call(
        body,
        grid=(grid,),
        in_specs=[
            pl.BlockSpec((Te, din_p), lambda i: (i, 0)),
            pl.BlockSpec((Te, 3), lambda i: (i, 0)),
            pl.BlockSpec((128, dio), lambda i: (0, 0)),
            pl.BlockSpec((8, 128), lambda i: (0, 0)),
            pl.BlockSpec((8, 128), lambda i: (0, 0)),
            pl.BlockSpec((8, 128), lambda i: (0, 0)),
            pl.BlockSpec((din_p, dio), lambda i: (0, 0)),
            pl.BlockSpec((dio, dout_p), lambda i: (0, 0)),
        ],
        out_specs=pl.BlockSpec((Te, dout_p), lambda i: (i, 0)),
        out_shape=jax.ShapeDtypeStruct((E, dout_p), jnp.float32),
    )


def _spline_parts(h, src2d, dst2d, attr, W):
    N, din = h.shape
    dout = W.shape[2]
    E = attr.shape[0]
    din_p = max(din, 16)
    dout_p = max(dout, 16)
    table = h if din == din_p else jnp.pad(h, ((0, 0), (0, din_p - din)))
    xj = _sc_gather(E, N, din_p)(table, src2d)
    wflat = jnp.pad(W.reshape(125, din * dout), ((0, 3), (0, 0)))
    consts = _edge_consts(din, din_p, dout, dout_p)
    oute = _tc_edge(E, din, din_p, dout, dout_p)(xj, attr, wflat, *consts)
    zeros = jnp.zeros((N, dout_p), jnp.float32)
    return _sc_scatter(E, N, dout_p)(oute, dst2d, zeros)


@functools.lru_cache(maxsize=None)
def _tc_node_pool(M, n, din, dout):
    def body(p_ref, x_ref, root_ref, bias_ref, pool_ref, o_ref):
        agg = p_ref[0] + p_ref[1]
        o = (agg + jnp.dot(x_ref[...], root_ref[...],
                           preferred_element_type=jnp.float32)
             + bias_ref[...])
        h = jnp.where(o > 0.0, o, jnp.exp(jnp.minimum(o, 0.0)) - 1.0)
        o_ref[...] = jnp.dot(pool_ref[0], h,
                             preferred_element_type=jnp.float32)[None]

    return pl.pallas_call(
        body,
        grid=(B,),
        in_specs=[
            pl.BlockSpec((2, n, dout), lambda b: (0, b, 0)),
            pl.BlockSpec((n, din), lambda b: (b, 0)),
            pl.BlockSpec((din, dout), lambda b: (0, 0)),
            pl.BlockSpec((1, dout), lambda b: (0, 0)),
            pl.BlockSpec((1, M, n), lambda b: (b, 0, 0)),
        ],
        out_specs=pl.BlockSpec((1, M, dout), lambda b: (b, 0, 0)),
        out_shape=jax.ShapeDtypeStruct((B, M, dout), jnp.float32),
    )


def _node_pool(P, parts, x, root, bias):
    Bb, M, n = P.shape
    dout = parts.shape[2]
    return _tc_node_pool(M, n, x.shape[1], dout)(
        parts, x, root, bias.reshape(1, dout), P)


@functools.lru_cache(maxsize=None)
def _tc_node(N, din, dout, dout_p):
    Tn = min(N, 4096)
    grid = N // Tn

    def body(p_ref, x_ref, root_ref, bias_ref, out_ref):
        agg = p_ref[0] + p_ref[1]
        o = (agg[:, :dout]
             + jnp.dot(x_ref[...], root_ref[...],
                       preferred_element_type=jnp.float32)
             + bias_ref[...])
        out_ref[...] = jnp.where(o > 0.0, o, jnp.exp(jnp.minimum(o, 0.0)) - 1.0)

    return pl.pallas_call(
        body,
        grid=(grid,),
        in_specs=[
            pl.BlockSpec((2, Tn, dout_p), lambda i: (0, i, 0)),
            pl.BlockSpec((Tn, din), lambda i: (i, 0)),
            pl.BlockSpec((din, dout), lambda i: (0, 0)),
            pl.BlockSpec((1, dout), lambda i: (0, 0)),
        ],
        out_specs=pl.BlockSpec((Tn, dout), lambda i: (i, 0)),
        out_shape=jax.ShapeDtypeStruct((N, dout), jnp.float32),
    )


def _middle(h, eps, fce1_w, fce1_b, fce21_w, fce21_b, fce22_w, fce22_b,
            fcd3_w, fcd3_b, fcd4_w, fcd4_b, P1n):
    f32 = jnp.float32

    def body(h_ref, eps_ref, w1_ref, b1_ref, w21_ref, b21_ref, w22_ref,
             b22_ref, w3_ref, b3_ref, w4_ref, b4_ref, p1n_ref,
             d0_ref, mu_ref, lv_ref):
        def fc(v, w, bb):
            return jnp.dot(v, w[...], preferred_element_type=f32) + bb[...]

        def elu(o):
            return jnp.where(o > 0.0, o, jnp.exp(jnp.minimum(o, 0.0)) - 1.0)

        h1 = elu(fc(h_ref[...], w1_ref, b1_ref))
        mu = fc(h1, w21_ref, b21_ref)
        lv = fc(h1, w22_ref, b22_ref)
        mu_ref[...] = mu
        lv_ref[...] = lv
        z = eps_ref[...] * jnp.exp(0.5 * lv) + mu
        d = elu(fc(z, w3_ref, b3_ref))
        d2 = elu(fc(d, w4_ref, b4_ref))
        for b in range(B):
            col = p1n_ref[:, b:b + 1]
            d0_ref[pl.ds(b * 8, 8), :] = col * d2[b:b + 1, :]

    return pl.pallas_call(
        body,
        out_shape=(jax.ShapeDtypeStruct((B * 8, 32), f32),
                   jax.ShapeDtypeStruct((B, 16), f32),
                   jax.ShapeDtypeStruct((B, 16), f32)),
    )(h, eps, fce1_w.T, fce1_b.reshape(1, -1), fce21_w.T,
      fce21_b.reshape(1, -1), fce22_w.T, fce22_b.reshape(1, -1),
      fcd3_w.T, fcd3_b.reshape(1, -1), fcd4_w.T, fcd4_b.reshape(1, -1),
      P1n[:, :, 0].T)


def kernel(x, edge_attr, bg1_edge_attr, bg2_edge_attr, bg3_edge_attr, bg4_edge_attr, P01, P12, P23, P34, Pn1, P1n, P10, P21, P32, P43, eps, W1, R1, b1, W2, R2, b2, W3, R3, b3, W4, R4, b4, W5, R5, b5, fce1_w, fce1_b, fce21_w, fce21_b, fce22_w, fce22_b, fcd3_w, fcd3_b, fcd4_w, fcd4_b, dW5, dR5, db5, dW4, dR4, db4, dW3, dR3, db3, dW2, dR2, db2, dW1, dR1, db1, edge_index, bg1_edge_index, bg2_edge_index, bg3_edge_index, bg4_edge_index):
    eis = []
    for ei in (edge_index, bg1_edge_index, bg2_edge_index, bg3_edge_index,
               bg4_edge_index):
        eis.append((ei[0].reshape(-1, 128), ei[1].reshape(-1, 128)))
    attrs = (edge_attr, bg1_edge_attr, bg2_edge_attr, bg3_edge_attr,
             bg4_edge_attr)

    h = x
    parts = _spline_parts(h, *eis[0], attrs[0], W1)
    h = _node_pool(P01, parts, h, R1, b1).reshape(-1, 16)
    parts = _spline_parts(h, *eis[1], attrs[1], W2)
    h = _node_pool(P12, parts, h, R2, b2).reshape(-1, 16)
    parts = _spline_parts(h, *eis[2], attrs[2], W3)
    h = _node_pool(P23, parts, h, R3, b3).reshape(-1, 16)
    parts = _spline_parts(h, *eis[3], attrs[3], W4)
    h = _node_pool(P34, parts, h, R4, b4).reshape(-1, 32)
    parts = _spline_parts(h, *eis[4], attrs[4], W5)
    h = _node_pool(Pn1, parts, h, R5, b5).reshape(B, 32)
    d, mu, logvar = _middle(h, eps, fce1_w, fce1_b, fce21_w, fce21_b,
                            fce22_w, fce22_b, fcd3_w, fcd3_b, fcd4_w, fcd4_b,
                            P1n)
    parts = _spline_parts(d, *eis[4], attrs[4], dW5)
    d = _node_pool(P43, parts, d, dR5, db5).reshape(-1, 32)
    parts = _spline_parts(d, *eis[3], attrs[3], dW4)
    d = _node_pool(P32, parts, d, dR4, db4).reshape(-1, 16)
    parts = _spline_parts(d, *eis[2], attrs[2], dW3)
    d = _node_pool(P21, parts, d, dR3, db3).reshape(-1, 16)
    parts = _spline_parts(d, *eis[1], attrs[1], dW2)
    d = _node_pool(P10, parts, d, dR2, db2).reshape(-1, 16)
    parts = _spline_parts(d, *eis[0], attrs[0], dW1)
    d = _tc_node(32768, 16, 3, 16)(parts, d, dR1, db1.reshape(1, 3))
    recon = d.reshape(B, -1)
    return (recon, mu, logvar)


# speedup vs baseline: 48.6747x; 1.4899x over previous
import functools

import numpy as np
import jax
import jax.numpy as jnp
from jax import lax
from jax.experimental import pallas as pl
from jax.experimental.pallas import tpu as pltpu, tpu_sc as plsc

K = 5
B = 16
NC, NS = 2, 16
NW = NC * NS


@functools.lru_cache(maxsize=None)
def _sc_gather(E, N, D, row):
    total_rows = E // 128
    rpw = max(1, total_rows // NW)
    active = total_rows // rpw
    CH = min(rpw, 16 if D <= 16 else 8)
    n_super = rpw // CH
    mesh = plsc.VectorSubcoreMesh(core_axis_name="c", subcore_axis_name="s")

    @functools.partial(
        pl.kernel,
        out_type=jax.ShapeDtypeStruct((E, D), jnp.float32),
        mesh=mesh,
        scratch_types=[
            pltpu.VMEM((2, CH, 128), jnp.int32),
            pltpu.VMEM((2, CH * 128, D), jnp.float32),
            pltpu.SemaphoreType.DMA,
            pltpu.SemaphoreType.DMA,
        ],
        compiler_params=pltpu.CompilerParams(use_tc_tiling_on_sc=False),
    )
    def kern(table, idx, out, idx_v, rows_v, gsem, wsem):
        wid = lax.axis_index("s") * NC + lax.axis_index("c")

        @pl.when(wid < active)
        def _():
            wbs = [None, None]
            for i in range(n_super):
                b = i & 1
                if wbs[b] is not None:
                    wbs[b].wait()
                r0 = wid * rpw + i * CH
                pltpu.sync_copy(idx.at[row, pl.ds(r0, CH)], idx_v.at[b])
                cps = [
                    pltpu.async_copy(
                        table.at[idx_v.at[b, j]],
                        rows_v.at[b, pl.ds(j * 128, 128)], gsem)
                    for j in range(CH)
                ]
                for cp in cps:
                    cp.wait()
                wbs[b] = pltpu.async_copy(
                    rows_v.at[b], out.at[pl.ds(r0 * 128, CH * 128)], wsem)
            for wb in wbs:
                if wb is not None:
                    wb.wait()

    return kern


@functools.lru_cache(maxsize=None)
def _sc_scatter(E, N, D, row):
    total_rows = E // 128
    rpw = max(1, total_rows // NW)
    active = total_rows // rpw
    CH = min(rpw, 16 if D <= 16 else 8)
    n_super = rpw // CH
    rows_t = N // NS
    mesh = plsc.VectorSubcoreMesh(core_axis_name="c", subcore_axis_name="s")

    @functools.partial(
        pl.kernel,
        out_type=jax.ShapeDtypeStruct((2, N, D), jnp.float32),
        mesh=mesh,
        scratch_types=[
            pltpu.VMEM((2, CH, 128), jnp.int32),
            pltpu.VMEM((2, CH * 128, D), jnp.float32),
            pltpu.SemaphoreType.DMA,
            pltpu.SemaphoreType.DMA,
            pltpu.VMEM_SHARED((N, D), jnp.float32),
        ],
        compiler_params=pltpu.CompilerParams(use_tc_tiling_on_sc=False),
    )
    def kern(upd, idx, zeros, out, idx_v, upd_v, ssem, psem, shared):
        c = lax.axis_index("c")
        s = lax.axis_index("s")
        wid = s * NC + c
        pltpu.sync_copy(zeros.at[pl.ds(s * rows_t, rows_t)],
                        shared.at[pl.ds(s * rows_t, rows_t)])
        plsc.subcore_barrier()

        @pl.when(wid < active)
        def _():
            def stage(i):
                b = i & 1
                r0 = wid * rpw + i * CH
                return (
                    pltpu.async_copy(idx.at[row, pl.ds(r0, CH)], idx_v.at[b],
                                     psem),
                    pltpu.async_copy(upd.at[pl.ds(r0 * 128, CH * 128)],
                                     upd_v.at[b], psem),
                )

            nxt = stage(0)
            for i in range(n_super):
                b = i & 1
                for cp in nxt:
                    cp.wait()
                if i + 1 < n_super:
                    nxt = stage(i + 1)
                cps = [
                    pltpu.async_copy(
                        upd_v.at[b, pl.ds(j * 128, 128)],
                        shared.at[idx_v.at[b, j]], ssem, add=True)
                    for j in range(CH)
                ]
                for cp in cps:
                    cp.wait()

        plsc.subcore_barrier()
        pltpu.sync_copy(shared.at[pl.ds(s * rows_t, rows_t)],
                        out.at[c, pl.ds(s * rows_t, rows_t)])

    return kern


@functools.lru_cache(maxsize=None)
def _edge_consts(din, din_p, dout, dout_p):
    spread = np.zeros((3, 24), np.float32)
    for d in range(3):
        spread[d, d * 8:(d + 1) * 8] = 1.0
    rep = []
    for d in range(3):
        m = np.zeros((24, 128), np.float32)
        for k in range(125):
            dig = (k, k // 5, k // 25)[d] % 5
            m[d * 8 + dig, k] = 1.0
        rep.append(m)
    repx = np.zeros((din_p, din * dout), np.float32)
    red = np.zeros((din * dout, dout_p), np.float32)
    for i in range(din):
        repx[i, i * dout:(i + 1) * dout] = 1.0
        for o in range(dout):
            red[i * dout + o, o] = 1.0
    return spread, rep[0], rep[1], rep[2], repx, red


@functools.lru_cache(maxsize=None)
def _tc_edge(E, din, din_p, dout, dout_p):
    Te = E // 32 if E >= 32768 else min(E, 2048)
    grid = E // Te
    dio = din * dout

    def body(xj_ref, attr_ref, wf_ref, sp_ref, r0_ref, r1_ref, r2_ref,
             rx_ref, rd_ref, out_ref):
        f32 = jnp.float32
        a = attr_ref[...]
        v = jnp.minimum(jnp.clip(a, 0.0, 1.0) * (K - 1), K - 1 - 1e-6)
        bot = jnp.floor(v)
        frac = v - bot
        botS = jnp.dot(bot, sp_ref[...], preferred_element_type=f32)
        fracS = jnp.dot(frac, sp_ref[...], preferred_element_type=f32)
        ii = lax.broadcasted_iota(jnp.int32, (Te, 24), 1)
        iiF = jnp.bitwise_and(ii, 7).astype(f32)
        csa = (jnp.where(iiF == botS, 1.0 - fracS, 0.0)
               + jnp.where(iiF == botS + 1.0, fracS, 0.0))
        C = (jnp.dot(csa, r0_ref[...], preferred_element_type=f32)
             * jnp.dot(csa, r1_ref[...], preferred_element_type=f32)
             * jnp.dot(csa, r2_ref[...], preferred_element_type=f32))
        weff = jnp.dot(C, wf_ref[...], preferred_element_type=f32)
        xrep = jnp.dot(xj_ref[...], rx_ref[...], preferred_element_type=f32)
        out_ref[...] = jnp.dot(xrep * weff, rd_ref[...],
                               preferred_element_type=f32)

    return pl.pallas_call(
        body,
        grid=(grid,),
        in_specs=[
            pl.BlockSpec((Te, din_p), lambda i: (i, 0)),
            pl.BlockSpec((Te, 3), lambda i: (i, 0)),
            pl.BlockSpec((128, dio), lambda i: (0, 0)),
            pl.BlockSpec((3, 24), lambda i: (0, 0)),
            pl.BlockSpec((24, 128), lambda i: (0, 0)),
            pl.BlockSpec((24, 128), lambda i: (0, 0)),
            pl.BlockSpec((24, 128), lambda i: (0, 0)),
            pl.BlockSpec((din_p, dio), lambda i: (0, 0)),
            pl.BlockSpec((dio, dout_p), lambda i: (0, 0)),
        ],
        out_specs=pl.BlockSpec((Te, dout_p), lambda i: (i, 0)),
        out_shape=jax.ShapeDtypeStruct((E, dout_p), jnp.float32),
    )


def _spline_parts(h, ei3, attr, W):
    N, din = h.shape
    dout = W.shape[2]
    E = attr.shape[0]
    din_p = max(din, 16)
    dout_p = max(dout, 16)
    table = h if din == din_p else jnp.pad(h, ((0, 0), (0, din_p - din)))
    xj = _sc_gather(E, N, din_p, 0)(table, ei3)
    wflat = jnp.pad(W.reshape(125, din * dout), ((0, 3), (0, 0)))
    consts = _edge_consts(din, din_p, dout, dout_p)
    oute = _tc_edge(E, din, din_p, dout, dout_p)(xj, attr, wflat, *consts)
    zeros = jnp.zeros((N, dout_p), jnp.float32)
    return _sc_scatter(E, N, dout_p, 1)(oute, ei3, zeros)


@functools.lru_cache(maxsize=None)
def _tc_node_pool(M, n, din, dout):
    def body(p_ref, x_ref, root_ref, bias_ref, pool_ref, o_ref):
        agg = p_ref[0] + p_ref[1]
        o = (agg + jnp.dot(x_ref[...], root_ref[...],
                           preferred_element_type=jnp.float32)
             + bias_ref[...])
        h = jnp.where(o > 0.0, o, jnp.exp(jnp.minimum(o, 0.0)) - 1.0)
        o_ref[...] = jnp.dot(pool_ref[0], h,
                             preferred_element_type=jnp.float32)[None]

    return pl.pallas_call(
        body,
        grid=(B,),
        in_specs=[
            pl.BlockSpec((2, n, dout), lambda b: (0, b, 0)),
            pl.BlockSpec((n, din), lambda b: (b, 0)),
            pl.BlockSpec((din, dout), lambda b: (0, 0)),
            pl.BlockSpec((1, dout), lambda b: (0, 0)),
            pl.BlockSpec((1, M, n), lambda b: (b, 0, 0)),
        ],
        out_specs=pl.BlockSpec((1, M, dout), lambda b: (b, 0, 0)),
        out_shape=jax.ShapeDtypeStruct((B, M, dout), jnp.float32),
    )


def _node_pool(P, parts, x, root, bias):
    Bb, M, n = P.shape
    dout = parts.shape[2]
    return _tc_node_pool(M, n, x.shape[1], dout)(
        parts, x, root, bias.reshape(1, dout), P)


@functools.lru_cache(maxsize=None)
def _tc_node(N, din, dout, dout_p):
    Tn = min(N, 4096)
    grid = N // Tn

    def body(p_ref, x_ref, root_ref, bias_ref, out_ref):
        agg = p_ref[0] + p_ref[1]
        o = (agg[:, :dout]
             + jnp.dot(x_ref[...], root_ref[...],
                       preferred_element_type=jnp.float32)
             + bias_ref[...])
        out_ref[...] = jnp.where(o > 0.0, o, jnp.exp(jnp.minimum(o, 0.0)) - 1.0)

    return pl.pallas_call(
        body,
        grid=(grid,),
        in_specs=[
            pl.BlockSpec((2, Tn, dout_p), lambda i: (0, i, 0)),
            pl.BlockSpec((Tn, din), lambda i: (i, 0)),
            pl.BlockSpec((din, dout), lambda i: (0, 0)),
            pl.BlockSpec((1, dout), lambda i: (0, 0)),
        ],
        out_specs=pl.BlockSpec((Tn, dout), lambda i: (i, 0)),
        out_shape=jax.ShapeDtypeStruct((N, dout), jnp.float32),
    )


def _middle(h, eps, fce1_w, fce1_b, fce21_w, fce21_b, fce22_w, fce22_b,
            fcd3_w, fcd3_b, fcd4_w, fcd4_b, P1n):
    f32 = jnp.float32

    def body(h_ref, eps_ref, w1_ref, b1_ref, w21_ref, b21_ref, w22_ref,
             b22_ref, w3_ref, b3_ref, w4_ref, b4_ref, p1n_ref,
             d0_ref, mu_ref, lv_ref):
        def fc(v, w, bb):
            return jnp.dot(v, w[...], preferred_element_type=f32) + bb[...]

        def elu(o):
            return jnp.where(o > 0.0, o, jnp.exp(jnp.minimum(o, 0.0)) - 1.0)

        h1 = elu(fc(h_ref[...], w1_ref, b1_ref))
        mu = fc(h1, w21_ref, b21_ref)
        lv = fc(h1, w22_ref, b22_ref)
        mu_ref[...] = mu
        lv_ref[...] = lv
        z = eps_ref[...] * jnp.exp(0.5 * lv) + mu
        d = elu(fc(z, w3_ref, b3_ref))
        d2 = elu(fc(d, w4_ref, b4_ref))
        for b in range(B):
            col = p1n_ref[:, b:b + 1]
            d0_ref[pl.ds(b * 8, 8), :] = col * d2[b:b + 1, :]

    return pl.pallas_call(
        body,
        out_shape=(jax.ShapeDtypeStruct((B * 8, 32), f32),
                   jax.ShapeDtypeStruct((B, 16), f32),
                   jax.ShapeDtypeStruct((B, 16), f32)),
    )(h, eps, fce1_w.T, fce1_b.reshape(1, -1), fce21_w.T,
      fce21_b.reshape(1, -1), fce22_w.T, fce22_b.reshape(1, -1),
      fcd3_w.T, fcd3_b.reshape(1, -1), fcd4_w.T, fcd4_b.reshape(1, -1),
      P1n[:, :, 0].T)


def kernel(x, edge_attr, bg1_edge_attr, bg2_edge_attr, bg3_edge_attr, bg4_edge_attr, P01, P12, P23, P34, Pn1, P1n, P10, P21, P32, P43, eps, W1, R1, b1, W2, R2, b2, W3, R3, b3, W4, R4, b4, W5, R5, b5, fce1_w, fce1_b, fce21_w, fce21_b, fce22_w, fce22_b, fcd3_w, fcd3_b, fcd4_w, fcd4_b, dW5, dR5, db5, dW4, dR4, db4, dW3, dR3, db3, dW2, dR2, db2, dW1, dR1, db1, edge_index, bg1_edge_index, bg2_edge_index, bg3_edge_index, bg4_edge_index):
    eis = []
    for ei in (edge_index, bg1_edge_index, bg2_edge_index, bg3_edge_index,
               bg4_edge_index):
        eis.append(ei.reshape(2, -1, 128))
    attrs = (edge_attr, bg1_edge_attr, bg2_edge_attr, bg3_edge_attr,
             bg4_edge_attr)

    h = x
    parts = _spline_parts(h, eis[0], attrs[0], W1)
    h = _node_pool(P01, parts, h, R1, b1).reshape(-1, 16)
    parts = _spline_parts(h, eis[1], attrs[1], W2)
    h = _node_pool(P12, parts, h, R2, b2).reshape(-1, 16)
    parts = _spline_parts(h, eis[2], attrs[2], W3)
    h = _node_pool(P23, parts, h, R3, b3).reshape(-1, 16)
    parts = _spline_parts(h, eis[3], attrs[3], W4)
    h = _node_pool(P34, parts, h, R4, b4).reshape(-1, 32)
    parts = _spline_parts(h, eis[4], attrs[4], W5)
    h = _node_pool(Pn1, parts, h, R5, b5).reshape(B, 32)
    d, mu, logvar = _middle(h, eps, fce1_w, fce1_b, fce21_w, fce21_b,
                            fce22_w, fce22_b, fcd3_w, fcd3_b, fcd4_w, fcd4_b,
                            P1n)
    parts = _spline_parts(d, eis[4], attrs[4], dW5)
    d = _node_pool(P43, parts, d, dR5, db5).reshape(-1, 32)
    parts = _spline_parts(d, eis[3], attrs[3], dW4)
    d = _node_pool(P32, parts, d, dR4, db4).reshape(-1, 16)
    parts = _spline_parts(d, eis[2], attrs[2], dW3)
    d = _node_pool(P21, parts, d, dR3, db3).reshape(-1, 16)
    parts = _spline_parts(d, eis[1], attrs[1], dW2)
    d = _node_pool(P10, parts, d, dR2, db2).reshape(-1, 16)
    parts = _spline_parts(d, eis[0], attrs[0], dW1)
    d = _tc_node(32768, 16, 3, 16)(parts, d, dR1, db1.reshape(1, 3))
    recon = d.reshape(B, -1)
    return (recon, mu, logvar)


# speedup vs baseline: 49.0361x; 1.0074x over previous
import functools

import numpy as np
import jax
import jax.numpy as jnp
from jax import lax
from jax.experimental import pallas as pl
from jax.experimental.pallas import tpu as pltpu, tpu_sc as plsc

K = 5
B = 16
NC, NS = 2, 16
NW = NC * NS


@functools.lru_cache(maxsize=None)
def _sc_gather(E, N, D, row):
    total_rows = E // 128
    rpw = max(1, total_rows // NW)
    active = total_rows // rpw
    CH = min(rpw, 16 if D <= 16 else 8)
    n_super = rpw // CH
    rows_tt = N // NS
    mesh = plsc.VectorSubcoreMesh(core_axis_name="c", subcore_axis_name="s")

    @functools.partial(
        pl.kernel,
        out_type=jax.ShapeDtypeStruct((E, D), jnp.float32),
        mesh=mesh,
        scratch_types=[
            pltpu.VMEM((2, CH, 128), jnp.int32),
            pltpu.VMEM((2, CH * 128, D), jnp.float32),
            pltpu.SemaphoreType.DMA,
            pltpu.SemaphoreType.DMA,
            pltpu.VMEM_SHARED((N, D), jnp.float32),
        ],
        compiler_params=pltpu.CompilerParams(use_tc_tiling_on_sc=False),
    )
    def kern(table, idx, out, idx_v, rows_v, gsem, wsem, shared):
        s = lax.axis_index("s")
        wid = s * NC + lax.axis_index("c")
        pltpu.sync_copy(table.at[pl.ds(s * rows_tt, rows_tt)],
                        shared.at[pl.ds(s * rows_tt, rows_tt)])
        plsc.subcore_barrier()

        @pl.when(wid < active)
        def _():
            wbs = [None, None]
            for i in range(n_super):
                b = i & 1
                if wbs[b] is not None:
                    wbs[b].wait()
                r0 = wid * rpw + i * CH
                pltpu.sync_copy(idx.at[row, pl.ds(r0, CH)], idx_v.at[b])
                cps = [
                    pltpu.async_copy(
                        shared.at[idx_v.at[b, j]],
                        rows_v.at[b, pl.ds(j * 128, 128)], gsem)
                    for j in range(CH)
                ]
                for cp in cps:
                    cp.wait()
                wbs[b] = pltpu.async_copy(
                    rows_v.at[b], out.at[pl.ds(r0 * 128, CH * 128)], wsem)
            for wb in wbs:
                if wb is not None:
                    wb.wait()

    return kern


@functools.lru_cache(maxsize=None)
def _sc_scatter(E, N, D, row):
    total_rows = E // 128
    rpw = max(1, total_rows // NW)
    active = total_rows // rpw
    CH = min(rpw, 16 if D <= 16 else 8)
    n_super = rpw // CH
    rows_t = N // NS
    mesh = plsc.VectorSubcoreMesh(core_axis_name="c", subcore_axis_name="s")

    @functools.partial(
        pl.kernel,
        out_type=jax.ShapeDtypeStruct((2, N, D), jnp.float32),
        mesh=mesh,
        scratch_types=[
            pltpu.VMEM((2, CH, 128), jnp.int32),
            pltpu.VMEM((2, CH * 128, D), jnp.float32),
            pltpu.SemaphoreType.DMA,
            pltpu.SemaphoreType.DMA,
            pltpu.VMEM_SHARED((N, D), jnp.float32),
        ],
        compiler_params=pltpu.CompilerParams(use_tc_tiling_on_sc=False),
    )
    def kern(upd, idx, zeros, out, idx_v, upd_v, ssem, psem, shared):
        c = lax.axis_index("c")
        s = lax.axis_index("s")
        wid = s * NC + c
        pltpu.sync_copy(zeros.at[pl.ds(s * rows_t, rows_t)],
                        shared.at[pl.ds(s * rows_t, rows_t)])
        plsc.subcore_barrier()

        @pl.when(wid < active)
        def _():
            def stage(i):
                b = i & 1
                r0 = wid * rpw + i * CH
                return (
                    pltpu.async_copy(idx.at[row, pl.ds(r0, CH)], idx_v.at[b],
                                     psem),
                    pltpu.async_copy(upd.at[pl.ds(r0 * 128, CH * 128)],
                                     upd_v.at[b], psem),
                )

            nxt = stage(0)
            for i in range(n_super):
                b = i & 1
                for cp in nxt:
                    cp.wait()
                if i + 1 < n_super:
                    nxt = stage(i + 1)
                cps = [
                    pltpu.async_copy(
                        upd_v.at[b, pl.ds(j * 128, 128)],
                        shared.at[idx_v.at[b, j]], ssem, add=True)
                    for j in range(CH)
                ]
                for cp in cps:
                    cp.wait()

        plsc.subcore_barrier()
        pltpu.sync_copy(shared.at[pl.ds(s * rows_t, rows_t)],
                        out.at[c, pl.ds(s * rows_t, rows_t)])

    return kern


@functools.lru_cache(maxsize=None)
def _edge_consts(din, din_p, dout, dout_p):
    spread = np.zeros((3, 24), np.float32)
    for d in range(3):
        spread[d, d * 8:(d + 1) * 8] = 1.0
    rep = []
    for d in range(3):
        m = np.zeros((24, 128), np.float32)
        for k in range(125):
            dig = (k, k // 5, k // 25)[d] % 5
            m[d * 8 + dig, k] = 1.0
        rep.append(m)
    repx = np.zeros((din_p, din * dout), np.float32)
    red = np.zeros((din * dout, dout_p), np.float32)
    for i in range(din):
        repx[i, i * dout:(i + 1) * dout] = 1.0
        for o in range(dout):
            red[i * dout + o, o] = 1.0
    return spread, rep[0], rep[1], rep[2], repx, red


@functools.lru_cache(maxsize=None)
def _tc_edge(E, din, din_p, dout, dout_p):
    Te = E // 32 if E >= 32768 else min(E, 2048)
    grid = E // Te
    dio = din * dout

    def body(xj_ref, attr_ref, wf_ref, sp_ref, r0_ref, r1_ref, r2_ref,
             rx_ref, rd_ref, out_ref):
        f32 = jnp.float32
        a = attr_ref[...]
        v = jnp.minimum(jnp.clip(a, 0.0, 1.0) * (K - 1), K - 1 - 1e-6)
        bot = jnp.floor(v)
        frac = v - bot
        botS = jnp.dot(bot, sp_ref[...], preferred_element_type=f32)
        fracS = jnp.dot(frac, sp_ref[...], preferred_element_type=f32)
        ii = lax.broadcasted_iota(jnp.int32, (Te, 24), 1)
        iiF = jnp.bitwise_and(ii, 7).astype(f32)
        csa = (jnp.where(iiF == botS, 1.0 - fracS, 0.0)
               + jnp.where(iiF == botS + 1.0, fracS, 0.0))
        C = (jnp.dot(csa, r0_ref[...], preferred_element_type=f32)
             * jnp.dot(csa, r1_ref[...], preferred_element_type=f32)
             * jnp.dot(csa, r2_ref[...], preferred_element_type=f32))
        weff = jnp.dot(C, wf_ref[...], preferred_element_type=f32)
        xrep = jnp.dot(xj_ref[...], rx_ref[...], preferred_element_type=f32)
        out_ref[...] = jnp.dot(xrep * weff, rd_ref[...],
                               preferred_element_type=f32)

    return pl.pallas_call(
        body,
        grid=(grid,),
        in_specs=[
            pl.BlockSpec((Te, din_p), lambda i: (i, 0)),
            pl.BlockSpec((Te, 3), lambda i: (i, 0)),
            pl.BlockSpec((128, dio), lambda i: (0, 0)),
            pl.BlockSpec((3, 24), lambda i: (0, 0)),
            pl.BlockSpec((24, 128), lambda i: (0, 0)),
            pl.BlockSpec((24, 128), lambda i: (0, 0)),
            pl.BlockSpec((24, 128), lambda i: (0, 0)),
            pl.BlockSpec((din_p, dio), lambda i: (0, 0)),
            pl.BlockSpec((dio, dout_p), lambda i: (0, 0)),
        ],
        out_specs=pl.BlockSpec((Te, dout_p), lambda i: (i, 0)),
        out_shape=jax.ShapeDtypeStruct((E, dout_p), jnp.float32),
    )


def _spline_parts(h, ei3, attr, W):
    N, din = h.shape
    dout = W.shape[2]
    E = attr.shape[0]
    din_p = max(din, 16)
    dout_p = max(dout, 16)
    table = h if din == din_p else jnp.pad(h, ((0, 0), (0, din_p - din)))
    xj = _sc_gather(E, N, din_p, 0)(table, ei3)
    wflat = jnp.pad(W.reshape(125, din * dout), ((0, 3), (0, 0)))
    consts = _edge_consts(din, din_p, dout, dout_p)
    oute = _tc_edge(E, din, din_p, dout, dout_p)(xj, attr, wflat, *consts)
    zeros = jnp.zeros((N, dout_p), jnp.float32)
    return _sc_scatter(E, N, dout_p, 1)(oute, ei3, zeros)


@functools.lru_cache(maxsize=None)
def _tc_node_pool(M, n, din, dout):
    def body(p_ref, x_ref, root_ref, bias_ref, pool_ref, o_ref):
        agg = p_ref[0] + p_ref[1]
        o = (agg + jnp.dot(x_ref[...], root_ref[...],
                           preferred_element_type=jnp.float32)
             + bias_ref[...])
        h = jnp.where(o > 0.0, o, jnp.exp(jnp.minimum(o, 0.0)) - 1.0)
        o_ref[...] = jnp.dot(pool_ref[0], h,
                             preferred_element_type=jnp.float32)[None]

    return pl.pallas_call(
        body,
        grid=(B,),
        in_specs=[
            pl.BlockSpec((2, n, dout), lambda b: (0, b, 0)),
            pl.BlockSpec((n, din), lambda b: (b, 0)),
            pl.BlockSpec((din, dout), lambda b: (0, 0)),
            pl.BlockSpec((1, dout), lambda b: (0, 0)),
            pl.BlockSpec((1, M, n), lambda b: (b, 0, 0)),
        ],
        out_specs=pl.BlockSpec((1, M, dout), lambda b: (b, 0, 0)),
        out_shape=jax.ShapeDtypeStruct((B, M, dout), jnp.float32),
    )


def _node_pool(P, parts, x, root, bias):
    Bb, M, n = P.shape
    dout = parts.shape[2]
    return _tc_node_pool(M, n, x.shape[1], dout)(
        parts, x, root, bias.reshape(1, dout), P)


@functools.lru_cache(maxsize=None)
def _tc_node(N, din, dout, dout_p):
    Tn = min(N, 4096)
    grid = N // Tn

    def body(p_ref, x_ref, root_ref, bias_ref, out_ref):
        agg = p_ref[0] + p_ref[1]
        o = (agg[:, :dout]
             + jnp.dot(x_ref[...], root_ref[...],
                       preferred_element_type=jnp.float32)
             + bias_ref[...])
        out_ref[...] = jnp.where(o > 0.0, o, jnp.exp(jnp.minimum(o, 0.0)) - 1.0)

    return pl.pallas_call(
        body,
        grid=(grid,),
        in_specs=[
            pl.BlockSpec((2, Tn, dout_p), lambda i: (0, i, 0)),
            pl.BlockSpec((Tn, din), lambda i: (i, 0)),
            pl.BlockSpec((din, dout), lambda i: (0, 0)),
            pl.BlockSpec((1, dout), lambda i: (0, 0)),
        ],
        out_specs=pl.BlockSpec((Tn, dout), lambda i: (i, 0)),
        out_shape=jax.ShapeDtypeStruct((N, dout), jnp.float32),
    )


def _middle(h, eps, fce1_w, fce1_b, fce21_w, fce21_b, fce22_w, fce22_b,
            fcd3_w, fcd3_b, fcd4_w, fcd4_b, P1n):
    f32 = jnp.float32

    def body(h_ref, eps_ref, w1_ref, b1_ref, w21_ref, b21_ref, w22_ref,
             b22_ref, w3_ref, b3_ref, w4_ref, b4_ref, p1n_ref,
             d0_ref, mu_ref, lv_ref):
        def fc(v, w, bb):
            return jnp.dot(v, w[...], preferred_element_type=f32) + bb[...]

        def elu(o):
            return jnp.where(o > 0.0, o, jnp.exp(jnp.minimum(o, 0.0)) - 1.0)

        h1 = elu(fc(h_ref[...], w1_ref, b1_ref))
        mu = fc(h1, w21_ref, b21_ref)
        lv = fc(h1, w22_ref, b22_ref)
        mu_ref[...] = mu
        lv_ref[...] = lv
        z = eps_ref[...] * jnp.exp(0.5 * lv) + mu
        d = elu(fc(z, w3_ref, b3_ref))
        d2 = elu(fc(d, w4_ref, b4_ref))
        for b in range(B):
            col = p1n_ref[:, b:b + 1]
            d0_ref[pl.ds(b * 8, 8), :] = col * d2[b:b + 1, :]

    return pl.pallas_call(
        body,
        out_shape=(jax.ShapeDtypeStruct((B * 8, 32), f32),
                   jax.ShapeDtypeStruct((B, 16), f32),
                   jax.ShapeDtypeStruct((B, 16), f32)),
    )(h, eps, fce1_w.T, fce1_b.reshape(1, -1), fce21_w.T,
      fce21_b.reshape(1, -1), fce22_w.T, fce22_b.reshape(1, -1),
      fcd3_w.T, fcd3_b.reshape(1, -1), fcd4_w.T, fcd4_b.reshape(1, -1),
      P1n[:, :, 0].T)


def kernel(x, edge_attr, bg1_edge_attr, bg2_edge_attr, bg3_edge_attr, bg4_edge_attr, P01, P12, P23, P34, Pn1, P1n, P10, P21, P32, P43, eps, W1, R1, b1, W2, R2, b2, W3, R3, b3, W4, R4, b4, W5, R5, b5, fce1_w, fce1_b, fce21_w, fce21_b, fce22_w, fce22_b, fcd3_w, fcd3_b, fcd4_w, fcd4_b, dW5, dR5, db5, dW4, dR4, db4, dW3, dR3, db3, dW2, dR2, db2, dW1, dR1, db1, edge_index, bg1_edge_index, bg2_edge_index, bg3_edge_index, bg4_edge_index):
    eis = []
    for ei in (edge_index, bg1_edge_index, bg2_edge_index, bg3_edge_index,
               bg4_edge_index):
        eis.append(ei.reshape(2, -1, 128))
    attrs = (edge_attr, bg1_edge_attr, bg2_edge_attr, bg3_edge_attr,
             bg4_edge_attr)

    h = x
    parts = _spline_parts(h, eis[0], attrs[0], W1)
    h = _node_pool(P01, parts, h, R1, b1).reshape(-1, 16)
    parts = _spline_parts(h, eis[1], attrs[1], W2)
    h = _node_pool(P12, parts, h, R2, b2).reshape(-1, 16)
    parts = _spline_parts(h, eis[2], attrs[2], W3)
    h = _node_pool(P23, parts, h, R3, b3).reshape(-1, 16)
    parts = _spline_parts(h, eis[3], attrs[3], W4)
    h = _node_pool(P34, parts, h, R4, b4).reshape(-1, 32)
    parts = _spline_parts(h, eis[4], attrs[4], W5)
    h = _node_pool(Pn1, parts, h, R5, b5).reshape(B, 32)
    d, mu, logvar = _middle(h, eps, fce1_w, fce1_b, fce21_w, fce21_b,
                            fce22_w, fce22_b, fcd3_w, fcd3_b, fcd4_w, fcd4_b,
                            P1n)
    parts = _spline_parts(d, eis[4], attrs[4], dW5)
    d = _node_pool(P43, parts, d, dR5, db5).reshape(-1, 32)
    parts = _spline_parts(d, eis[3], attrs[3], dW4)
    d = _node_pool(P32, parts, d, dR4, db4).reshape(-1, 16)
    parts = _spline_parts(d, eis[2], attrs[2], dW3)
    d = _node_pool(P21, parts, d, dR3, db3).reshape(-1, 16)
    parts = _spline_parts(d, eis[1], attrs[1], dW2)
    d = _node_pool(P10, parts, d, dR2, db2).reshape(-1, 16)
    parts = _spline_parts(d, eis[0], attrs[0], dW1)
    d = _tc_node(32768, 16, 3, 16)(parts, d, dR1, db1.reshape(1, 3))
    recon = d.reshape(B, -1)
    return (recon, mu, logvar)
